# Initial kernel scaffold; baseline (speedup 1.0000x reference)
import jax
import jax.numpy as jnp
from jax.experimental import pallas as pl


def kernel(x, edge_index, W1, b1, W2, b2):
    raise NotImplementedError("write your pallas kernel here")


# speedup vs baseline: 7.4545x; 7.4545x over previous
import functools

import jax
import jax.numpy as jnp
from jax import lax
from jax.experimental import pallas as pl
from jax.experimental.pallas import tpu as pltpu
from jax.experimental.pallas import tpu_sc as plsc

N = 10000
E = 320000
D = 128

NC = 2
NS = 16
NW = NC * NS

CH = 128
CPW = 80
E_PAD = NW * CPW * CH
NCHUNK = E_PAD // CH

NPAD = 10240
ROWS_PER_TILE = NPAD // NS

BLK = 1024
GRID = NPAD // BLK

_mesh = plsc.VectorSubcoreMesh(core_axis_name="c", subcore_axis_name="s")


@functools.partial(
    pl.kernel,
    out_type=jax.ShapeDtypeStruct((NC, NPAD, D), jnp.float32),
    mesh=_mesh,
    scratch_types=[
        pltpu.VMEM((CPW, CH), jnp.int32),
        pltpu.VMEM((CH, D), jnp.float32),
        pltpu.VMEM_SHARED((NPAD, D), jnp.float32),
    ],
)
def _sc_degree(dst_hbm, ones_hbm, zeros_hbm, out_hbm, dst_v, ones_v, acc):
    c = lax.axis_index("c")
    s = lax.axis_index("s")
    wid = c * NS + s
    pltpu.sync_copy(zeros_hbm.at[pl.ds(s * ROWS_PER_TILE, ROWS_PER_TILE)],
                    acc.at[pl.ds(s * ROWS_PER_TILE, ROWS_PER_TILE)])
    pltpu.sync_copy(dst_hbm.at[pl.ds(wid * CPW, CPW)], dst_v)
    pltpu.sync_copy(ones_hbm, ones_v)
    plsc.subcore_barrier()

    def body(j, carry):
        pltpu.sync_copy(ones_v, acc.at[dst_v.at[j]], add=True)
        return carry

    lax.fori_loop(0, CPW, body, 0)
    plsc.subcore_barrier()
    pltpu.sync_copy(acc.at[pl.ds(s * ROWS_PER_TILE, ROWS_PER_TILE)],
                    out_hbm.at[c].at[pl.ds(s * ROWS_PER_TILE, ROWS_PER_TILE)])


@functools.partial(
    pl.kernel,
    out_type=jax.ShapeDtypeStruct((NC, NPAD, D), jnp.float32),
    mesh=_mesh,
    scratch_types=[
        pltpu.VMEM((CPW, CH), jnp.int32),
        pltpu.VMEM((CPW, CH), jnp.int32),
        pltpu.VMEM((CH, D), jnp.float32),
        pltpu.VMEM_SHARED((NPAD, D), jnp.float32),
        pltpu.SemaphoreType.DMA,
    ],
)
def _sc_mp(y_hbm, src_hbm, dst_hbm, zeros_hbm, out_hbm,
           src_v, dst_v, rows_v, acc, sem):
    c = lax.axis_index("c")
    s = lax.axis_index("s")
    wid = c * NS + s
    pltpu.sync_copy(zeros_hbm.at[pl.ds(s * ROWS_PER_TILE, ROWS_PER_TILE)],
                    acc.at[pl.ds(s * ROWS_PER_TILE, ROWS_PER_TILE)])
    pltpu.sync_copy(src_hbm.at[pl.ds(wid * CPW, CPW)], src_v)
    pltpu.sync_copy(dst_hbm.at[pl.ds(wid * CPW, CPW)], dst_v)
    plsc.subcore_barrier()

    def body(j, carry):
        pltpu.async_copy(y_hbm.at[src_v.at[j]], rows_v, sem).wait()
        pltpu.sync_copy(rows_v, acc.at[dst_v.at[j]], add=True)
        return carry

    lax.fori_loop(0, CPW, body, 0)
    plsc.subcore_barrier()
    pltpu.sync_copy(acc.at[pl.ds(s * ROWS_PER_TILE, ROWS_PER_TILE)],
                    out_hbm.at[c].at[pl.ds(s * ROWS_PER_TILE, ROWS_PER_TILE)])


def _tc_prep_body(degp_ref, x_ref, w_ref, y_ref, dinv_ref):
    deg = jnp.sum(degp_ref[0, :, :1] + degp_ref[1, :, :1], axis=1) + 1.0
    dinv = lax.rsqrt(deg)
    dinv_ref[...] = dinv[:, None]
    xw = jnp.dot(x_ref[...], w_ref[...], preferred_element_type=jnp.float32,
                 precision=lax.Precision.HIGHEST)
    y_ref[...] = xw * dinv[:, None]


def _tc_mid_body(p_ref, y_ref, dinv_ref, b_ref, w_ref, y2_ref):
    dinv = dinv_ref[...]
    h = dinv * (p_ref[0] + p_ref[1] + y_ref[...]) + b_ref[...]
    h = jnp.maximum(h, 0.0)
    hw = jnp.dot(h, w_ref[...], preferred_element_type=jnp.float32,
                 precision=lax.Precision.HIGHEST)
    y2_ref[...] = hw * dinv


def _tc_post_body(p_ref, y_ref, dinv_ref, b_ref, out_ref):
    out_ref[...] = dinv_ref[...] * (p_ref[0] + p_ref[1] + y_ref[...]) + b_ref[...]


def _tc_prep(degp, xp, W1):
    return pl.pallas_call(
        _tc_prep_body,
        grid=(GRID,),
        in_specs=[
            pl.BlockSpec((NC, BLK, D), lambda i: (0, i, 0)),
            pl.BlockSpec((BLK, D), lambda i: (i, 0)),
            pl.BlockSpec((D, D), lambda i: (0, 0)),
        ],
        out_specs=[
            pl.BlockSpec((BLK, D), lambda i: (i, 0)),
            pl.BlockSpec((BLK, 1), lambda i: (i, 0)),
        ],
        out_shape=[
            jax.ShapeDtypeStruct((NPAD, D), jnp.float32),
            jax.ShapeDtypeStruct((NPAD, 1), jnp.float32),
        ],
    )(degp, xp, W1)


def _tc_mid(p1, y1, dinv, b1, W2):
    return pl.pallas_call(
        _tc_mid_body,
        grid=(GRID,),
        in_specs=[
            pl.BlockSpec((NC, BLK, D), lambda i: (0, i, 0)),
            pl.BlockSpec((BLK, D), lambda i: (i, 0)),
            pl.BlockSpec((BLK, 1), lambda i: (i, 0)),
            pl.BlockSpec((1, D), lambda i: (0, 0)),
            pl.BlockSpec((D, D), lambda i: (0, 0)),
        ],
        out_specs=pl.BlockSpec((BLK, D), lambda i: (i, 0)),
        out_shape=jax.ShapeDtypeStruct((NPAD, D), jnp.float32),
    )(p1, y1, dinv, b1, W2)


def _tc_post(p2, y2, dinv, b2):
    return pl.pallas_call(
        _tc_post_body,
        grid=(GRID,),
        in_specs=[
            pl.BlockSpec((NC, BLK, D), lambda i: (0, i, 0)),
            pl.BlockSpec((BLK, D), lambda i: (i, 0)),
            pl.BlockSpec((BLK, 1), lambda i: (i, 0)),
            pl.BlockSpec((1, D), lambda i: (0, 0)),
        ],
        out_specs=pl.BlockSpec((BLK, D), lambda i: (i, 0)),
        out_shape=jax.ShapeDtypeStruct((NPAD, D), jnp.float32),
    )(p2, y2, dinv, b2)


def kernel(x, edge_index, W1, b1, W2, b2):
    src = edge_index[0].astype(jnp.int32)
    dst = edge_index[1].astype(jnp.int32)
    srcp = jnp.concatenate(
        [src, jnp.zeros((E_PAD - E,), jnp.int32)]).reshape(NCHUNK, CH)
    dstp = jnp.concatenate(
        [dst, jnp.full((E_PAD - E,), N, jnp.int32)]).reshape(NCHUNK, CH)
    xp = jnp.pad(x, ((0, NPAD - N), (0, 0)))
    zeros_d = jnp.zeros((NPAD, D), jnp.float32)
    ones_d = jnp.ones((CH, D), jnp.float32)

    degp = _sc_degree(dstp, ones_d, zeros_d)
    y1, dinv = _tc_prep(degp, xp, W1)
    p1 = _sc_mp(y1, srcp, dstp, zeros_d)
    y2 = _tc_mid(p1, y1, dinv, b1.reshape(1, D), W2)
    p2 = _sc_mp(y2, srcp, dstp, zeros_d)
    out = _tc_post(p2, y2, dinv, b2.reshape(1, D))
    return out[:N]


# speedup vs baseline: 8.1339x; 1.0911x over previous
import functools

import jax
import jax.numpy as jnp
from jax import lax
from jax.experimental import pallas as pl
from jax.experimental.pallas import tpu as pltpu
from jax.experimental.pallas import tpu_sc as plsc

N = 10000
E = 320000
D = 128

NC = 2
NS = 16
NW = NC * NS

CH = 128
CPW = 80
E_PAD = NW * CPW * CH
NCHUNK = E_PAD // CH
KPB = 8
NBLK = CPW // KPB

NPAD = 10112
ROWS_PER_TILE = NPAD // NS

BLK = 1264
GRID = NPAD // BLK

_mesh = plsc.VectorSubcoreMesh(core_axis_name="c", subcore_axis_name="s")


@functools.partial(
    pl.kernel,
    out_type=jax.ShapeDtypeStruct((NC, NPAD, D), jnp.float32),
    mesh=_mesh,
    scratch_types=[
        pltpu.VMEM((CPW, CH), jnp.int32),
        pltpu.VMEM((CH, D), jnp.float32),
        pltpu.VMEM_SHARED((NPAD, D), jnp.float32),
    ],
)
def _sc_degree(dst_hbm, ones_hbm, zeros_hbm, out_hbm, dst_v, ones_v, acc):
    c = lax.axis_index("c")
    s = lax.axis_index("s")
    wid = c * NS + s
    pltpu.sync_copy(zeros_hbm.at[pl.ds(s * ROWS_PER_TILE, ROWS_PER_TILE)],
                    acc.at[pl.ds(s * ROWS_PER_TILE, ROWS_PER_TILE)])
    pltpu.sync_copy(dst_hbm.at[pl.ds(wid * CPW, CPW)], dst_v)
    pltpu.sync_copy(ones_hbm, ones_v)
    plsc.subcore_barrier()

    def body(j, carry):
        pltpu.sync_copy(ones_v, acc.at[dst_v.at[j]], add=True)
        return carry

    lax.fori_loop(0, CPW, body, 0)
    plsc.subcore_barrier()
    pltpu.sync_copy(acc.at[pl.ds(s * ROWS_PER_TILE, ROWS_PER_TILE)],
                    out_hbm.at[c].at[pl.ds(s * ROWS_PER_TILE, ROWS_PER_TILE)])


@functools.partial(
    pl.kernel,
    out_type=jax.ShapeDtypeStruct((NC, NPAD, D), jnp.float32),
    mesh=_mesh,
    scratch_types=[
        pltpu.VMEM((2, KPB, CH), jnp.int32),
        pltpu.VMEM((2, CH, D), jnp.float32),
        pltpu.VMEM_SHARED((NPAD, D), jnp.float32),
        pltpu.SemaphoreType.DMA,
        pltpu.SemaphoreType.DMA,
    ],
)
def _sc_mp(y_hbm, src_hbm, dst_hbm, zeros_hbm, out_hbm,
           idx_v, rows_v, acc, sem_g, sem_s):
    c = lax.axis_index("c")
    s = lax.axis_index("s")
    wid = c * NS + s
    pltpu.sync_copy(zeros_hbm.at[pl.ds(s * ROWS_PER_TILE, ROWS_PER_TILE)],
                    acc.at[pl.ds(s * ROWS_PER_TILE, ROWS_PER_TILE)])
    plsc.subcore_barrier()

    def body(blk, carry):
        rowbase = wid * CPW + blk * KPB
        pltpu.sync_copy(src_hbm.at[pl.ds(rowbase, KPB)], idx_v.at[0])
        pltpu.sync_copy(dst_hbm.at[pl.ds(rowbase, KPB)], idx_v.at[1])
        g = {}
        sc = {}
        g[0] = pltpu.async_copy(y_hbm.at[idx_v.at[0].at[0]],
                                rows_v.at[0], sem_g)
        for k in range(KPB):
            if k + 1 < KPB:
                if k >= 1:
                    sc[k - 1].wait()
                g[k + 1] = pltpu.async_copy(y_hbm.at[idx_v.at[0].at[k + 1]],
                                            rows_v.at[(k + 1) % 2], sem_g)
            g[k].wait()
            sc[k] = pltpu.async_copy(rows_v.at[k % 2],
                                     acc.at[idx_v.at[1].at[k]],
                                     sem_s, add=True)
        sc[KPB - 2].wait()
        sc[KPB - 1].wait()
        return carry

    lax.fori_loop(0, NBLK, body, 0)
    plsc.subcore_barrier()
    pltpu.sync_copy(acc.at[pl.ds(s * ROWS_PER_TILE, ROWS_PER_TILE)],
                    out_hbm.at[c].at[pl.ds(s * ROWS_PER_TILE, ROWS_PER_TILE)])


def _tc_prep_body(degp_ref, x_ref, w_ref, y_ref, dinv_ref):
    deg = jnp.sum(degp_ref[0, :, :1] + degp_ref[1, :, :1], axis=1) + 1.0
    dinv = lax.rsqrt(deg)
    dinv_ref[...] = dinv[:, None]
    xw = jnp.dot(x_ref[...], w_ref[...], preferred_element_type=jnp.float32,
                 precision=lax.Precision.HIGHEST)
    y_ref[...] = xw * dinv[:, None]


def _tc_mid_body(p_ref, y_ref, dinv_ref, b_ref, w_ref, y2_ref):
    dinv = dinv_ref[...]
    h = dinv * (p_ref[0] + p_ref[1] + y_ref[...]) + b_ref[...]
    h = jnp.maximum(h, 0.0)
    hw = jnp.dot(h, w_ref[...], preferred_element_type=jnp.float32,
                 precision=lax.Precision.HIGHEST)
    y2_ref[...] = hw * dinv


def _tc_post_body(p_ref, y_ref, dinv_ref, b_ref, out_ref):
    out_ref[...] = dinv_ref[...] * (p_ref[0] + p_ref[1] + y_ref[...]) + b_ref[...]


def _tc_prep(degp, xp, W1):
    return pl.pallas_call(
        _tc_prep_body,
        grid=(GRID,),
        in_specs=[
            pl.BlockSpec((NC, BLK, D), lambda i: (0, i, 0)),
            pl.BlockSpec((BLK, D), lambda i: (i, 0)),
            pl.BlockSpec((D, D), lambda i: (0, 0)),
        ],
        out_specs=[
            pl.BlockSpec((BLK, D), lambda i: (i, 0)),
            pl.BlockSpec((BLK, 1), lambda i: (i, 0)),
        ],
        out_shape=[
            jax.ShapeDtypeStruct((NPAD, D), jnp.float32),
            jax.ShapeDtypeStruct((NPAD, 1), jnp.float32),
        ],
    )(degp, xp, W1)


def _tc_mid(p1, y1, dinv, b1, W2):
    return pl.pallas_call(
        _tc_mid_body,
        grid=(GRID,),
        in_specs=[
            pl.BlockSpec((NC, BLK, D), lambda i: (0, i, 0)),
            pl.BlockSpec((BLK, D), lambda i: (i, 0)),
            pl.BlockSpec((BLK, 1), lambda i: (i, 0)),
            pl.BlockSpec((1, D), lambda i: (0, 0)),
            pl.BlockSpec((D, D), lambda i: (0, 0)),
        ],
        out_specs=pl.BlockSpec((BLK, D), lambda i: (i, 0)),
        out_shape=jax.ShapeDtypeStruct((NPAD, D), jnp.float32),
    )(p1, y1, dinv, b1, W2)


def _tc_post(p2, y2, dinv, b2):
    return pl.pallas_call(
        _tc_post_body,
        grid=(GRID,),
        in_specs=[
            pl.BlockSpec((NC, BLK, D), lambda i: (0, i, 0)),
            pl.BlockSpec((BLK, D), lambda i: (i, 0)),
            pl.BlockSpec((BLK, 1), lambda i: (i, 0)),
            pl.BlockSpec((1, D), lambda i: (0, 0)),
        ],
        out_specs=pl.BlockSpec((BLK, D), lambda i: (i, 0)),
        out_shape=jax.ShapeDtypeStruct((NPAD, D), jnp.float32),
    )(p2, y2, dinv, b2)


def kernel(x, edge_index, W1, b1, W2, b2):
    src = edge_index[0].astype(jnp.int32)
    dst = edge_index[1].astype(jnp.int32)
    srcp = jnp.concatenate(
        [src, jnp.zeros((E_PAD - E,), jnp.int32)]).reshape(NCHUNK, CH)
    dstp = jnp.concatenate(
        [dst, jnp.full((E_PAD - E,), N, jnp.int32)]).reshape(NCHUNK, CH)
    xp = jnp.pad(x, ((0, NPAD - N), (0, 0)))
    zeros_d = jnp.zeros((NPAD, D), jnp.float32)
    ones_d = jnp.ones((CH, D), jnp.float32)

    degp = _sc_degree(dstp, ones_d, zeros_d)
    y1, dinv = _tc_prep(degp, xp, W1)
    p1 = _sc_mp(y1, srcp, dstp, zeros_d)
    y2 = _tc_mid(p1, y1, dinv, b1.reshape(1, D), W2)
    p2 = _sc_mp(y2, srcp, dstp, zeros_d)
    out = _tc_post(p2, y2, dinv, b2.reshape(1, D))
    return out[:N]


# speedup vs baseline: 25.0409x; 3.0786x over previous
import functools

import jax
import jax.numpy as jnp
from jax import lax
from jax.experimental import pallas as pl
from jax.experimental.pallas import tpu as pltpu
from jax.experimental.pallas import tpu_sc as plsc

N = 10000
E = 320000
D = 128

NC = 2
NS = 16
NW = NC * NS

CH = 128
CPW = 80
E_PAD = NW * CPW * CH
NCHUNK = E_PAD // CH
KPB = 8
NBLK = CPW // KPB

NPAD = 10112
ROWS_PER_TILE = NPAD // NS

BLK = 1264
GRID = NPAD // BLK

_mesh = plsc.VectorSubcoreMesh(core_axis_name="c", subcore_axis_name="s")


@functools.partial(
    pl.kernel,
    out_type=jax.ShapeDtypeStruct((NC, NPAD, D), jnp.float32),
    mesh=_mesh,
    scratch_types=[
        pltpu.VMEM((CPW, CH), jnp.int32),
        pltpu.VMEM((CH, D), jnp.float32),
        pltpu.VMEM_SHARED((NPAD, D), jnp.float32),
    ],
)
def _sc_degree(dst_hbm, ones_hbm, zeros_hbm, out_hbm, dst_v, ones_v, acc):
    c = lax.axis_index("c")
    s = lax.axis_index("s")
    wid = c * NS + s
    pltpu.sync_copy(zeros_hbm.at[pl.ds(s * ROWS_PER_TILE, ROWS_PER_TILE)],
                    acc.at[pl.ds(s * ROWS_PER_TILE, ROWS_PER_TILE)])
    pltpu.sync_copy(dst_hbm.at[pl.ds(wid * CPW, CPW)], dst_v)
    pltpu.sync_copy(ones_hbm, ones_v)
    plsc.subcore_barrier()

    def body(j, carry):
        pltpu.sync_copy(ones_v, acc.at[dst_v.at[j]], add=True)
        return carry

    lax.fori_loop(0, CPW, body, 0)
    plsc.subcore_barrier()
    pltpu.sync_copy(acc.at[pl.ds(s * ROWS_PER_TILE, ROWS_PER_TILE)],
                    out_hbm.at[c].at[pl.ds(s * ROWS_PER_TILE, ROWS_PER_TILE)])


@functools.partial(
    pl.kernel,
    out_type=jax.ShapeDtypeStruct((NC, NPAD, D), jnp.float32),
    mesh=_mesh,
    scratch_types=[
        pltpu.VMEM((2, KPB, CH), jnp.int32),
        pltpu.VMEM((2, CH, D), jnp.float32),
        pltpu.VMEM_SHARED((NPAD, D), jnp.float32),
        pltpu.SemaphoreType.DMA,
        pltpu.SemaphoreType.DMA,
    ],
)
def _sc_mp(y_hbm, src_hbm, dst_hbm, zeros_hbm, out_hbm,
           idx_v, rows_v, acc, sem_g, sem_s):
    c = lax.axis_index("c")
    s = lax.axis_index("s")
    wid = c * NS + s
    pltpu.sync_copy(zeros_hbm.at[pl.ds(s * ROWS_PER_TILE, ROWS_PER_TILE)],
                    acc.at[pl.ds(s * ROWS_PER_TILE, ROWS_PER_TILE)])
    plsc.subcore_barrier()

    def body(blk, carry):
        rowbase = wid * CPW + blk * KPB
        pltpu.sync_copy(src_hbm.at[pl.ds(rowbase, KPB)], idx_v.at[0])
        pltpu.sync_copy(dst_hbm.at[pl.ds(rowbase, KPB)], idx_v.at[1])
        g = {}
        sc = {}
        g[0] = pltpu.async_copy(y_hbm.at[idx_v.at[0].at[0]],
                                rows_v.at[0], sem_g)
        for k in range(KPB):
            if k + 1 < KPB:
                if k >= 1:
                    sc[k - 1].wait()
                g[k + 1] = pltpu.async_copy(y_hbm.at[idx_v.at[0].at[k + 1]],
                                            rows_v.at[(k + 1) % 2], sem_g)
            g[k].wait()
            sc[k] = pltpu.async_copy(rows_v.at[k % 2],
                                     acc.at[idx_v.at[1].at[k]],
                                     sem_s, add=True)
        sc[KPB - 2].wait()
        sc[KPB - 1].wait()
        return carry

    lax.fori_loop(0, NBLK, body, 0)
    plsc.subcore_barrier()
    pltpu.sync_copy(acc.at[pl.ds(s * ROWS_PER_TILE, ROWS_PER_TILE)],
                    out_hbm.at[c].at[pl.ds(s * ROWS_PER_TILE, ROWS_PER_TILE)])


def _tc_prep_body(degp_ref, x_ref, w_ref, y_ref, dinv_ref):
    deg = jnp.sum(degp_ref[0, :, :1] + degp_ref[1, :, :1], axis=1) + 1.0
    dinv = lax.rsqrt(deg)
    dinv_ref[...] = dinv[:, None]
    xw = jnp.dot(x_ref[...], w_ref[...], preferred_element_type=jnp.float32,
                 precision=lax.Precision.HIGHEST)
    y_ref[...] = xw * dinv[:, None]


def _tc_mid_body(p_ref, y_ref, dinv_ref, b_ref, w_ref, y2_ref):
    dinv = dinv_ref[...]
    h = dinv * (p_ref[0] + p_ref[1] + y_ref[...]) + b_ref[...]
    h = jnp.maximum(h, 0.0)
    hw = jnp.dot(h, w_ref[...], preferred_element_type=jnp.float32,
                 precision=lax.Precision.HIGHEST)
    y2_ref[...] = hw * dinv


def _tc_post_body(p_ref, y_ref, dinv_ref, b_ref, out_ref):
    out_ref[...] = dinv_ref[...] * (p_ref[0] + p_ref[1] + y_ref[...]) + b_ref[...]


def _tc_prep(degp, xp, W1):
    return pl.pallas_call(
        _tc_prep_body,
        grid=(GRID,),
        in_specs=[
            pl.BlockSpec((NC, BLK, D), lambda i: (0, i, 0)),
            pl.BlockSpec((BLK, D), lambda i: (i, 0)),
            pl.BlockSpec((D, D), lambda i: (0, 0)),
        ],
        out_specs=[
            pl.BlockSpec((BLK, D), lambda i: (i, 0)),
            pl.BlockSpec((BLK, 1), lambda i: (i, 0)),
        ],
        out_shape=[
            jax.ShapeDtypeStruct((NPAD, D), jnp.float32),
            jax.ShapeDtypeStruct((NPAD, 1), jnp.float32),
        ],
    )(degp, xp, W1)


def _tc_mid(p1, y1, dinv, b1, W2):
    return pl.pallas_call(
        _tc_mid_body,
        grid=(GRID,),
        in_specs=[
            pl.BlockSpec((NC, BLK, D), lambda i: (0, i, 0)),
            pl.BlockSpec((BLK, D), lambda i: (i, 0)),
            pl.BlockSpec((BLK, 1), lambda i: (i, 0)),
            pl.BlockSpec((1, D), lambda i: (0, 0)),
            pl.BlockSpec((D, D), lambda i: (0, 0)),
        ],
        out_specs=pl.BlockSpec((BLK, D), lambda i: (i, 0)),
        out_shape=jax.ShapeDtypeStruct((NPAD, D), jnp.float32),
    )(p1, y1, dinv, b1, W2)


def _tc_post(p2, y2, dinv, b2):
    return pl.pallas_call(
        _tc_post_body,
        grid=(GRID,),
        in_specs=[
            pl.BlockSpec((NC, BLK, D), lambda i: (0, i, 0)),
            pl.BlockSpec((BLK, D), lambda i: (i, 0)),
            pl.BlockSpec((BLK, 1), lambda i: (i, 0)),
            pl.BlockSpec((1, D), lambda i: (0, 0)),
        ],
        out_specs=pl.BlockSpec((BLK, D), lambda i: (i, 0)),
        out_shape=jax.ShapeDtypeStruct((NPAD, D), jnp.float32),
    )(p2, y2, dinv, b2)


def kernel(x, edge_index, W1, b1, W2, b2):
    src = edge_index[0].astype(jnp.int32)
    dst = edge_index[1].astype(jnp.int32)
    pad_i = jnp.arange(E_PAD - E, dtype=jnp.int32)
    srcp = jnp.concatenate([src, pad_i % N]).reshape(NCHUNK, CH)
    dstp = jnp.concatenate(
        [dst, N + pad_i % (NPAD - N)]).reshape(NCHUNK, CH)
    xp = jnp.pad(x, ((0, NPAD - N), (0, 0)))
    zeros_d = jnp.zeros((NPAD, D), jnp.float32)
    ones_d = jnp.ones((CH, D), jnp.float32)

    degp = _sc_degree(dstp, ones_d, zeros_d)
    y1, dinv = _tc_prep(degp, xp, W1)
    p1 = _sc_mp(y1, srcp, dstp, zeros_d)
    y2 = _tc_mid(p1, y1, dinv, b1.reshape(1, D), W2)
    p2 = _sc_mp(y2, srcp, dstp, zeros_d)
    out = _tc_post(p2, y2, dinv, b2.reshape(1, D))
    return out[:N]


# speedup vs baseline: 29.4305x; 1.1753x over previous
import functools

import jax
import jax.numpy as jnp
from jax import lax
from jax.experimental import pallas as pl
from jax.experimental.pallas import tpu as pltpu
from jax.experimental.pallas import tpu_sc as plsc

N = 10000
E = 320000
D = 128

NC = 2
NS = 16
NW = NC * NS

CH = 128
CPW = 80
E_PAD = NW * CPW * CH
NCHUNK = E_PAD // CH
KPB = 8
NBLK = CPW // KPB

NPAD = 10240
ROWS_PER_TILE = NPAD // NS

BLK = 1280
GRID = NPAD // BLK

_mesh = plsc.VectorSubcoreMesh(core_axis_name="c", subcore_axis_name="s")


HR = NPAD // 128


@functools.partial(
    pl.kernel,
    out_type=jax.ShapeDtypeStruct((NW, HR, 128), jnp.float32),
    mesh=_mesh,
    compiler_params=pltpu.CompilerParams(needs_layout_passes=False),
    scratch_types=[
        pltpu.VMEM((CPW, CH), jnp.int32),
        pltpu.VMEM((HR, 128), jnp.float32),
    ],
)
def _sc_degree(dst_hbm, out_hbm, dst_v, hist):
    c = lax.axis_index("c")
    s = lax.axis_index("s")
    wid = c * NS + s
    pltpu.sync_copy(dst_hbm.at[pl.ds(wid * CPW, CPW)], dst_v)

    zero = jnp.zeros((16,), jnp.float32)

    def zbody(t, carry):
        hist[t >> 3, pl.ds((t & 7) * 16, 16)] = zero
        return carry

    lax.fori_loop(0, HR * 8, zbody, 0)

    ones = jnp.ones((16,), jnp.float32)

    def body(t, carry):
        idx = dst_v[t >> 3, pl.ds((t & 7) * 16, 16)]
        plsc.addupdate_scatter(hist, [idx >> 7, idx & 127], ones)
        return carry

    lax.fori_loop(0, CPW * CH // 16, body, 0)
    pltpu.sync_copy(hist, out_hbm.at[wid])


@functools.partial(
    pl.kernel,
    out_type=jax.ShapeDtypeStruct((NC, NPAD, D), jnp.float32),
    mesh=_mesh,
    scratch_types=[
        pltpu.VMEM((2, KPB, CH), jnp.int32),
        pltpu.VMEM((2, CH, D), jnp.float32),
        pltpu.VMEM_SHARED((NPAD, D), jnp.float32),
        pltpu.SemaphoreType.DMA,
        pltpu.SemaphoreType.DMA,
    ],
)
def _sc_mp(y_hbm, src_hbm, dst_hbm, zeros_hbm, out_hbm,
           idx_v, rows_v, acc, sem_g, sem_s):
    c = lax.axis_index("c")
    s = lax.axis_index("s")
    wid = c * NS + s
    pltpu.sync_copy(zeros_hbm.at[pl.ds(s * ROWS_PER_TILE, ROWS_PER_TILE)],
                    acc.at[pl.ds(s * ROWS_PER_TILE, ROWS_PER_TILE)])
    plsc.subcore_barrier()

    def body(blk, carry):
        rowbase = wid * CPW + blk * KPB
        pltpu.sync_copy(src_hbm.at[pl.ds(rowbase, KPB)], idx_v.at[0])
        pltpu.sync_copy(dst_hbm.at[pl.ds(rowbase, KPB)], idx_v.at[1])
        g = {}
        sc = {}
        g[0] = pltpu.async_copy(y_hbm.at[idx_v.at[0].at[0]],
                                rows_v.at[0], sem_g)
        for k in range(KPB):
            if k + 1 < KPB:
                if k >= 1:
                    sc[k - 1].wait()
                g[k + 1] = pltpu.async_copy(y_hbm.at[idx_v.at[0].at[k + 1]],
                                            rows_v.at[(k + 1) % 2], sem_g)
            g[k].wait()
            sc[k] = pltpu.async_copy(rows_v.at[k % 2],
                                     acc.at[idx_v.at[1].at[k]],
                                     sem_s, add=True)
        sc[KPB - 2].wait()
        sc[KPB - 1].wait()
        return carry

    lax.fori_loop(0, NBLK, body, 0)
    plsc.subcore_barrier()
    pltpu.sync_copy(acc.at[pl.ds(s * ROWS_PER_TILE, ROWS_PER_TILE)],
                    out_hbm.at[c].at[pl.ds(s * ROWS_PER_TILE, ROWS_PER_TILE)])


def _tc_mm1_body(x_ref, w_ref, xw_ref):
    xw_ref[...] = jnp.dot(x_ref[...], w_ref[...],
                          preferred_element_type=jnp.float32,
                          precision=lax.Precision.HIGHEST)


def _tc_prep_body(hist_ref, xw_ref, y_ref, dinv_ref):
    deg = jnp.sum(hist_ref[...], axis=0) + 1.0
    dinv = lax.rsqrt(deg)
    dinv_ref[...] = dinv[:, None]
    y_ref[...] = xw_ref[...] * dinv[:, None]


def _tc_mid_body(p_ref, y_ref, dinv_ref, b_ref, w_ref, y2_ref):
    dinv = dinv_ref[...]
    h = dinv * (p_ref[0] + p_ref[1] + y_ref[...]) + b_ref[...]
    h = jnp.maximum(h, 0.0)
    hw = jnp.dot(h, w_ref[...], preferred_element_type=jnp.float32,
                 precision=lax.Precision.HIGHEST)
    y2_ref[...] = hw * dinv


def _tc_post_body(p_ref, y_ref, dinv_ref, b_ref, out_ref):
    out_ref[...] = dinv_ref[...] * (p_ref[0] + p_ref[1] + y_ref[...]) + b_ref[...]


def _tc_mm1(xp, W1):
    return pl.pallas_call(
        _tc_mm1_body,
        grid=(GRID,),
        in_specs=[
            pl.BlockSpec((BLK, D), lambda i: (i, 0)),
            pl.BlockSpec((D, D), lambda i: (0, 0)),
        ],
        out_specs=pl.BlockSpec((BLK, D), lambda i: (i, 0)),
        out_shape=jax.ShapeDtypeStruct((NPAD, D), jnp.float32),
    )(xp, W1)


def _tc_prep(hist2d, xw1):
    return pl.pallas_call(
        _tc_prep_body,
        grid=(GRID,),
        in_specs=[
            pl.BlockSpec((NW, BLK), lambda i: (0, i)),
            pl.BlockSpec((BLK, D), lambda i: (i, 0)),
        ],
        out_specs=[
            pl.BlockSpec((BLK, D), lambda i: (i, 0)),
            pl.BlockSpec((BLK, 1), lambda i: (i, 0)),
        ],
        out_shape=[
            jax.ShapeDtypeStruct((NPAD, D), jnp.float32),
            jax.ShapeDtypeStruct((NPAD, 1), jnp.float32),
        ],
    )(hist2d, xw1)


def _tc_mid(p1, y1, dinv, b1, W2):
    return pl.pallas_call(
        _tc_mid_body,
        grid=(GRID,),
        in_specs=[
            pl.BlockSpec((NC, BLK, D), lambda i: (0, i, 0)),
            pl.BlockSpec((BLK, D), lambda i: (i, 0)),
            pl.BlockSpec((BLK, 1), lambda i: (i, 0)),
            pl.BlockSpec((1, D), lambda i: (0, 0)),
            pl.BlockSpec((D, D), lambda i: (0, 0)),
        ],
        out_specs=pl.BlockSpec((BLK, D), lambda i: (i, 0)),
        out_shape=jax.ShapeDtypeStruct((NPAD, D), jnp.float32),
    )(p1, y1, dinv, b1, W2)


def _tc_post(p2, y2, dinv, b2):
    return pl.pallas_call(
        _tc_post_body,
        grid=(GRID,),
        in_specs=[
            pl.BlockSpec((NC, BLK, D), lambda i: (0, i, 0)),
            pl.BlockSpec((BLK, D), lambda i: (i, 0)),
            pl.BlockSpec((BLK, 1), lambda i: (i, 0)),
            pl.BlockSpec((1, D), lambda i: (0, 0)),
        ],
        out_specs=pl.BlockSpec((BLK, D), lambda i: (i, 0)),
        out_shape=jax.ShapeDtypeStruct((NPAD, D), jnp.float32),
    )(p2, y2, dinv, b2)


def kernel(x, edge_index, W1, b1, W2, b2):
    src = edge_index[0].astype(jnp.int32)
    dst = edge_index[1].astype(jnp.int32)
    pad_i = jnp.arange(E_PAD - E, dtype=jnp.int32)
    srcp = jnp.concatenate([src, pad_i % N]).reshape(NCHUNK, CH)
    dstp = jnp.concatenate(
        [dst, N + pad_i % (NPAD - N)]).reshape(NCHUNK, CH)
    xp = jnp.pad(x, ((0, NPAD - N), (0, 0)))
    zeros_d = jnp.zeros((NPAD, D), jnp.float32)

    hist = _sc_degree(dstp)
    xw1 = _tc_mm1(xp, W1)
    y1, dinv = _tc_prep(hist.reshape(NW, NPAD), xw1)
    p1 = _sc_mp(y1, srcp, dstp, zeros_d)
    y2 = _tc_mid(p1, y1, dinv, b1.reshape(1, D), W2)
    p2 = _sc_mp(y2, srcp, dstp, zeros_d)
    out = _tc_post(p2, y2, dinv, b2.reshape(1, D))
    return out[:N]


# speedup vs baseline: 30.1680x; 1.0251x over previous
import functools

import jax
import jax.numpy as jnp
from jax import lax
from jax.experimental import pallas as pl
from jax.experimental.pallas import tpu as pltpu
from jax.experimental.pallas import tpu_sc as plsc

N = 10000
E = 320000
D = 128

NC = 2
NS = 16
NW = NC * NS

CH = 128
CPW = 80
E_PAD = NW * CPW * CH
NCHUNK = E_PAD // CH
KPB = 8
NBLK = CPW // KPB

NPAD = 10240
ROWS_PER_TILE = NPAD // NS

BLK = 1280
GRID = NPAD // BLK

_mesh = plsc.VectorSubcoreMesh(core_axis_name="c", subcore_axis_name="s")


HR = NPAD // 128


@functools.partial(
    pl.kernel,
    out_type=jax.ShapeDtypeStruct((NW, HR, 128), jnp.float32),
    mesh=_mesh,
    compiler_params=pltpu.CompilerParams(needs_layout_passes=False),
    scratch_types=[
        pltpu.VMEM((CPW, CH), jnp.int32),
        pltpu.VMEM((HR, 128), jnp.float32),
    ],
)
def _sc_degree(dst_hbm, out_hbm, dst_v, hist):
    c = lax.axis_index("c")
    s = lax.axis_index("s")
    wid = c * NS + s
    pltpu.sync_copy(dst_hbm.at[pl.ds(wid * CPW, CPW)], dst_v)

    zero = jnp.zeros((16,), jnp.float32)

    def zbody(t, carry):
        hist[t >> 3, pl.ds((t & 7) * 16, 16)] = zero
        return carry

    lax.fori_loop(0, HR * 8, zbody, 0)

    ones = jnp.ones((16,), jnp.float32)

    def body(t, carry):
        idx = dst_v[t >> 3, pl.ds((t & 7) * 16, 16)]
        plsc.addupdate_scatter(hist, [idx >> 7, idx & 127], ones)
        return carry

    lax.fori_loop(0, CPW * CH // 16, body, 0)
    pltpu.sync_copy(hist, out_hbm.at[wid])


@functools.partial(
    pl.kernel,
    out_type=jax.ShapeDtypeStruct((NC, NPAD, D), jnp.float32),
    mesh=_mesh,
    scratch_types=[
        pltpu.VMEM((2, KPB, CH), jnp.int32),
        pltpu.VMEM((2, CH, D), jnp.float32),
        pltpu.VMEM_SHARED((NPAD, D), jnp.float32),
        pltpu.SemaphoreType.DMA,
        pltpu.SemaphoreType.DMA,
    ],
)
def _sc_mp(y_hbm, src_hbm, dst_hbm, zeros_hbm, out_hbm,
           idx_v, rows_v, acc, sem_g, sem_s):
    c = lax.axis_index("c")
    s = lax.axis_index("s")
    wid = c * NS + s
    pltpu.sync_copy(zeros_hbm.at[pl.ds(s * ROWS_PER_TILE, ROWS_PER_TILE)],
                    acc.at[pl.ds(s * ROWS_PER_TILE, ROWS_PER_TILE)])
    plsc.subcore_barrier()

    def body(blk, carry):
        rowbase = wid * CPW + blk * KPB
        pltpu.sync_copy(src_hbm.at[pl.ds(rowbase, KPB)], idx_v.at[0])
        pltpu.sync_copy(dst_hbm.at[pl.ds(rowbase, KPB)], idx_v.at[1])
        def fire_gather(k):
            b = k % 2
            return [
                pltpu.async_copy(
                    y_hbm.at[idx_v.at[0, k, pl.ds(h * (CH // 2), CH // 2)]],
                    rows_v.at[b, pl.ds(h * (CH // 2), CH // 2)], sem_g)
                for h in range(2)
            ]

        g = {}
        sc = {}
        g[0] = fire_gather(0)
        for k in range(KPB):
            if k + 1 < KPB:
                if k >= 1:
                    sc[k - 1].wait()
                g[k + 1] = fire_gather(k + 1)
            for d in g[k]:
                d.wait()
            sc[k] = pltpu.async_copy(rows_v.at[k % 2],
                                     acc.at[idx_v.at[1].at[k]],
                                     sem_s, add=True)
        sc[KPB - 2].wait()
        sc[KPB - 1].wait()
        return carry

    lax.fori_loop(0, NBLK, body, 0)
    plsc.subcore_barrier()
    pltpu.sync_copy(acc.at[pl.ds(s * ROWS_PER_TILE, ROWS_PER_TILE)],
                    out_hbm.at[c].at[pl.ds(s * ROWS_PER_TILE, ROWS_PER_TILE)])


def _tc_mm1_body(x_ref, w_ref, xw_ref):
    xw_ref[...] = jnp.dot(x_ref[...], w_ref[...],
                          preferred_element_type=jnp.float32,
                          precision=lax.Precision.HIGHEST)


def _tc_prep_body(hist_ref, xw_ref, y_ref, dinv_ref):
    deg = jnp.sum(hist_ref[...], axis=0) + 1.0
    dinv = lax.rsqrt(deg)
    dinv_ref[...] = dinv[:, None]
    y_ref[...] = xw_ref[...] * dinv[:, None]


def _tc_mid_body(p_ref, y_ref, dinv_ref, b_ref, w_ref, y2_ref):
    dinv = dinv_ref[...]
    h = dinv * (p_ref[0] + p_ref[1] + y_ref[...]) + b_ref[...]
    h = jnp.maximum(h, 0.0)
    hw = jnp.dot(h, w_ref[...], preferred_element_type=jnp.float32,
                 precision=lax.Precision.HIGHEST)
    y2_ref[...] = hw * dinv


def _tc_post_body(p_ref, y_ref, dinv_ref, b_ref, out_ref):
    out_ref[...] = dinv_ref[...] * (p_ref[0] + p_ref[1] + y_ref[...]) + b_ref[...]


def _tc_mm1(xp, W1):
    return pl.pallas_call(
        _tc_mm1_body,
        grid=(GRID,),
        in_specs=[
            pl.BlockSpec((BLK, D), lambda i: (i, 0)),
            pl.BlockSpec((D, D), lambda i: (0, 0)),
        ],
        out_specs=pl.BlockSpec((BLK, D), lambda i: (i, 0)),
        out_shape=jax.ShapeDtypeStruct((NPAD, D), jnp.float32),
    )(xp, W1)


def _tc_prep(hist2d, xw1):
    return pl.pallas_call(
        _tc_prep_body,
        grid=(GRID,),
        in_specs=[
            pl.BlockSpec((NW, BLK), lambda i: (0, i)),
            pl.BlockSpec((BLK, D), lambda i: (i, 0)),
        ],
        out_specs=[
            pl.BlockSpec((BLK, D), lambda i: (i, 0)),
            pl.BlockSpec((BLK, 1), lambda i: (i, 0)),
        ],
        out_shape=[
            jax.ShapeDtypeStruct((NPAD, D), jnp.float32),
            jax.ShapeDtypeStruct((NPAD, 1), jnp.float32),
        ],
    )(hist2d, xw1)


def _tc_mid(p1, y1, dinv, b1, W2):
    return pl.pallas_call(
        _tc_mid_body,
        grid=(GRID,),
        in_specs=[
            pl.BlockSpec((NC, BLK, D), lambda i: (0, i, 0)),
            pl.BlockSpec((BLK, D), lambda i: (i, 0)),
            pl.BlockSpec((BLK, 1), lambda i: (i, 0)),
            pl.BlockSpec((1, D), lambda i: (0, 0)),
            pl.BlockSpec((D, D), lambda i: (0, 0)),
        ],
        out_specs=pl.BlockSpec((BLK, D), lambda i: (i, 0)),
        out_shape=jax.ShapeDtypeStruct((NPAD, D), jnp.float32),
    )(p1, y1, dinv, b1, W2)


def _tc_post(p2, y2, dinv, b2):
    return pl.pallas_call(
        _tc_post_body,
        grid=(GRID,),
        in_specs=[
            pl.BlockSpec((NC, BLK, D), lambda i: (0, i, 0)),
            pl.BlockSpec((BLK, D), lambda i: (i, 0)),
            pl.BlockSpec((BLK, 1), lambda i: (i, 0)),
            pl.BlockSpec((1, D), lambda i: (0, 0)),
        ],
        out_specs=pl.BlockSpec((BLK, D), lambda i: (i, 0)),
        out_shape=jax.ShapeDtypeStruct((N, D), jnp.float32),
    )(p2, y2, dinv, b2)


def kernel(x, edge_index, W1, b1, W2, b2):
    src = edge_index[0].astype(jnp.int32)
    dst = edge_index[1].astype(jnp.int32)
    pad_i = jnp.arange(E_PAD - E, dtype=jnp.int32)
    srcp = jnp.concatenate([src, pad_i % N]).reshape(NCHUNK, CH)
    dstp = jnp.concatenate(
        [dst, N + pad_i % (NPAD - N)]).reshape(NCHUNK, CH)
    zeros_d = jnp.zeros((NPAD, D), jnp.float32)

    hist = _sc_degree(dstp)
    xw1 = _tc_mm1(x, W1)
    y1, dinv = _tc_prep(hist.reshape(NW, NPAD), xw1)
    p1 = _sc_mp(y1, srcp, dstp, zeros_d)
    y2 = _tc_mid(p1, y1, dinv, b1.reshape(1, D), W2)
    p2 = _sc_mp(y2, srcp, dstp, zeros_d)
    return _tc_post(p2, y2, dinv, b2.reshape(1, D))


# speedup vs baseline: 32.1139x; 1.0645x over previous
import functools

import jax
import jax.numpy as jnp
from jax import lax
from jax.experimental import pallas as pl
from jax.experimental.pallas import tpu as pltpu
from jax.experimental.pallas import tpu_sc as plsc

N = 10000
E = 320000
D = 128

NC = 2
NS = 16
NW = NC * NS

CH = 64
CPW = 160
E_PAD = NW * CPW * CH
NCHUNK = E_PAD // CH
KPB = 16
NBLK = CPW // KPB
NBUF = 4

NPAD = 10240
ROWS_PER_TILE = NPAD // NS

BLK = 1280
GRID = NPAD // BLK

_mesh = plsc.VectorSubcoreMesh(core_axis_name="c", subcore_axis_name="s")


HR = NPAD // 128


@functools.partial(
    pl.kernel,
    out_type=jax.ShapeDtypeStruct((NW, HR, 128), jnp.float32),
    mesh=_mesh,
    compiler_params=pltpu.CompilerParams(needs_layout_passes=False),
    scratch_types=[
        pltpu.VMEM((CPW, CH), jnp.int32),
        pltpu.VMEM((HR, 128), jnp.float32),
    ],
)
def _sc_degree(dst_hbm, out_hbm, dst_v, hist):
    c = lax.axis_index("c")
    s = lax.axis_index("s")
    wid = c * NS + s
    pltpu.sync_copy(dst_hbm.at[pl.ds(wid * CPW, CPW)], dst_v)

    zero = jnp.zeros((16,), jnp.float32)

    def zbody(t, carry):
        hist[t >> 3, pl.ds((t & 7) * 16, 16)] = zero
        return carry

    lax.fori_loop(0, HR * 8, zbody, 0)

    ones = jnp.ones((16,), jnp.float32)

    def body(t, carry):
        idx = dst_v[t >> 2, pl.ds((t & 3) * 16, 16)]
        plsc.addupdate_scatter(hist, [idx >> 7, idx & 127], ones)
        return carry

    lax.fori_loop(0, CPW * CH // 16, body, 0)
    pltpu.sync_copy(hist, out_hbm.at[wid])


@functools.partial(
    pl.kernel,
    out_type=jax.ShapeDtypeStruct((NC, NPAD, D), jnp.float32),
    mesh=_mesh,
    scratch_types=[
        pltpu.VMEM((2, KPB, CH), jnp.int32),
        pltpu.VMEM((NBUF, CH, D), jnp.float32),
        pltpu.VMEM_SHARED((NPAD, D), jnp.float32),
        pltpu.SemaphoreType.DMA,
        pltpu.SemaphoreType.DMA,
    ],
)
def _sc_mp(y_hbm, src_hbm, dst_hbm, zeros_hbm, out_hbm,
           idx_v, rows_v, acc, sem_g, sem_s):
    c = lax.axis_index("c")
    s = lax.axis_index("s")
    wid = c * NS + s
    pltpu.sync_copy(zeros_hbm.at[pl.ds(s * ROWS_PER_TILE, ROWS_PER_TILE)],
                    acc.at[pl.ds(s * ROWS_PER_TILE, ROWS_PER_TILE)])
    plsc.subcore_barrier()

    def body(blk, carry):
        rowbase = wid * CPW + blk * KPB
        pltpu.sync_copy(src_hbm.at[pl.ds(rowbase, KPB)], idx_v.at[0])
        pltpu.sync_copy(dst_hbm.at[pl.ds(rowbase, KPB)], idx_v.at[1])
        def fire_gather(k):
            return pltpu.async_copy(y_hbm.at[idx_v.at[0, k]],
                                    rows_v.at[k % NBUF], sem_g)

        g = {}
        sc = {}
        for k in range(NBUF - 1):
            g[k] = fire_gather(k)
        for k in range(KPB):
            if k + NBUF - 1 < KPB:
                if k >= 1:
                    sc[k - 1].wait()
                g[k + NBUF - 1] = fire_gather(k + NBUF - 1)
            g[k].wait()
            sc[k] = pltpu.async_copy(rows_v.at[k % NBUF],
                                     acc.at[idx_v.at[1, k]],
                                     sem_s, add=True)
        for k in range(KPB - NBUF, KPB):
            sc[k].wait()
        return carry

    lax.fori_loop(0, NBLK, body, 0)
    plsc.subcore_barrier()
    pltpu.sync_copy(acc.at[pl.ds(s * ROWS_PER_TILE, ROWS_PER_TILE)],
                    out_hbm.at[c].at[pl.ds(s * ROWS_PER_TILE, ROWS_PER_TILE)])


def _tc_mm1_body(x_ref, w_ref, xw_ref):
    xw_ref[...] = jnp.dot(x_ref[...], w_ref[...],
                          preferred_element_type=jnp.float32,
                          precision=lax.Precision.HIGHEST)


def _tc_prep_body(hist_ref, xw_ref, y_ref, dinv_ref):
    deg = jnp.sum(hist_ref[...], axis=0) + 1.0
    dinv = lax.rsqrt(deg)
    dinv_ref[...] = dinv[:, None]
    y_ref[...] = xw_ref[...] * dinv[:, None]


def _tc_mid_body(p_ref, y_ref, dinv_ref, b_ref, w_ref, y2_ref):
    dinv = dinv_ref[...]
    h = dinv * (p_ref[0] + p_ref[1] + y_ref[...]) + b_ref[...]
    h = jnp.maximum(h, 0.0)
    hw = jnp.dot(h, w_ref[...], preferred_element_type=jnp.float32,
                 precision=lax.Precision.HIGHEST)
    y2_ref[...] = hw * dinv


def _tc_post_body(p_ref, y_ref, dinv_ref, b_ref, out_ref):
    out_ref[...] = dinv_ref[...] * (p_ref[0] + p_ref[1] + y_ref[...]) + b_ref[...]


def _tc_mm1(xp, W1):
    return pl.pallas_call(
        _tc_mm1_body,
        grid=(GRID,),
        in_specs=[
            pl.BlockSpec((BLK, D), lambda i: (i, 0)),
            pl.BlockSpec((D, D), lambda i: (0, 0)),
        ],
        out_specs=pl.BlockSpec((BLK, D), lambda i: (i, 0)),
        out_shape=jax.ShapeDtypeStruct((NPAD, D), jnp.float32),
    )(xp, W1)


def _tc_prep(hist2d, xw1):
    return pl.pallas_call(
        _tc_prep_body,
        grid=(GRID,),
        in_specs=[
            pl.BlockSpec((NW, BLK), lambda i: (0, i)),
            pl.BlockSpec((BLK, D), lambda i: (i, 0)),
        ],
        out_specs=[
            pl.BlockSpec((BLK, D), lambda i: (i, 0)),
            pl.BlockSpec((BLK, 1), lambda i: (i, 0)),
        ],
        out_shape=[
            jax.ShapeDtypeStruct((NPAD, D), jnp.float32),
            jax.ShapeDtypeStruct((NPAD, 1), jnp.float32),
        ],
    )(hist2d, xw1)


def _tc_mid(p1, y1, dinv, b1, W2):
    return pl.pallas_call(
        _tc_mid_body,
        grid=(GRID,),
        in_specs=[
            pl.BlockSpec((NC, BLK, D), lambda i: (0, i, 0)),
            pl.BlockSpec((BLK, D), lambda i: (i, 0)),
            pl.BlockSpec((BLK, 1), lambda i: (i, 0)),
            pl.BlockSpec((1, D), lambda i: (0, 0)),
            pl.BlockSpec((D, D), lambda i: (0, 0)),
        ],
        out_specs=pl.BlockSpec((BLK, D), lambda i: (i, 0)),
        out_shape=jax.ShapeDtypeStruct((NPAD, D), jnp.float32),
    )(p1, y1, dinv, b1, W2)


def _tc_post(p2, y2, dinv, b2):
    return pl.pallas_call(
        _tc_post_body,
        grid=(GRID,),
        in_specs=[
            pl.BlockSpec((NC, BLK, D), lambda i: (0, i, 0)),
            pl.BlockSpec((BLK, D), lambda i: (i, 0)),
            pl.BlockSpec((BLK, 1), lambda i: (i, 0)),
            pl.BlockSpec((1, D), lambda i: (0, 0)),
        ],
        out_specs=pl.BlockSpec((BLK, D), lambda i: (i, 0)),
        out_shape=jax.ShapeDtypeStruct((N, D), jnp.float32),
    )(p2, y2, dinv, b2)


def kernel(x, edge_index, W1, b1, W2, b2):
    src = edge_index[0].astype(jnp.int32)
    dst = edge_index[1].astype(jnp.int32)
    pad_i = jnp.arange(E_PAD - E, dtype=jnp.int32)
    srcp = jnp.concatenate([src, pad_i % N]).reshape(NCHUNK, CH)
    dstp = jnp.concatenate(
        [dst, N + pad_i % (NPAD - N)]).reshape(NCHUNK, CH)
    zeros_d = jnp.zeros((NPAD, D), jnp.float32)

    hist = _sc_degree(dstp)
    xw1 = _tc_mm1(x, W1)
    y1, dinv = _tc_prep(hist.reshape(NW, NPAD), xw1)
    p1 = _sc_mp(y1, srcp, dstp, zeros_d)
    y2 = _tc_mid(p1, y1, dinv, b1.reshape(1, D), W2)
    p2 = _sc_mp(y2, srcp, dstp, zeros_d)
    return _tc_post(p2, y2, dinv, b2.reshape(1, D))
